# baseline (speedup 1.0000x reference)
import functools

import jax
import jax.numpy as jnp
from jax import lax
from jax.experimental import pallas as pl
from jax.experimental.pallas import tpu as pltpu
from jax.experimental.pallas import tpu_sc as plsc

N_AGENTS = 1000000
N_BATCH = 16384
NC = 2
NS = 16
NW = NC * NS
L = 16

BPW = N_BATCH // NW
GCH = 128
NG = BPW // GCH
VPW = (N_AGENTS // L) // NW
MAIN = NW * VPW * L
NEXTRA = (N_AGENTS - MAIN) // L

_mesh = plsc.VectorSubcoreMesh(core_axis_name="c", subcore_axis_name="s")


def _abshalf_tanh(x):
    t = jnp.exp(-jnp.abs(x))
    return (1.0 - t) / (1.0 + t)


@functools.partial(
    pl.kernel,
    mesh=_mesh,
    out_type=(
        jax.ShapeDtypeStruct((N_BATCH,), jnp.float32),
        jax.ShapeDtypeStruct((NC, L), jnp.float32),
    ),
    scratch_types=[
        pltpu.VMEM((NG, GCH), jnp.int32),
        pltpu.VMEM((BPW,), jnp.float32),
        pltpu.VMEM((BPW,), jnp.float32),
        pltpu.VMEM((BPW,), jnp.float32),
        pltpu.VMEM((BPW,), jnp.float32),
        pltpu.VMEM((VPW * L,), jnp.float32),
        pltpu.VMEM((L,), jnp.float32),
        pltpu.VMEM((L,), jnp.float32),
        pltpu.VMEM((NS * L,), jnp.float32),
        pltpu.VMEM_SHARED((NS * L,), jnp.float32),
        pltpu.SemaphoreType.DMA,
        pltpu.SemaphoreType.DMA,
    ],
)
def _sc_doxastic(bel_hbm, gt_hbm, ids_hbm, log_hbm, cb_out, loss_out,
                 idx_v, gat_v, bel_v, gt_v, cb_v, chunk_v, extra_v, part_v,
                 all_v, shared, sem_g, sem_c):
    cid = lax.axis_index("c")
    sid = lax.axis_index("s")
    wid = cid * NS + sid
    b0 = wid * BPW

    pltpu.sync_copy(ids_hbm.at[wid], idx_v)
    gds = [
        pltpu.async_copy(
            log_hbm.at[idx_v.at[k]],
            gat_v.at[pl.ds(k * GCH, GCH)],
            sem_g,
        )
        for k in range(NG)
    ]
    cd = pltpu.async_copy(log_hbm.at[pl.ds(wid * VPW * L, VPW * L)],
                          chunk_v, sem_c)
    pltpu.sync_copy(bel_hbm.at[pl.ds(b0, BPW)], bel_v)
    pltpu.sync_copy(gt_hbm.at[pl.ds(b0, BPW)], gt_v)
    eoff = MAIN + (wid % NEXTRA) * L
    pltpu.sync_copy(log_hbm.at[pl.ds(eoff, L)], extra_v)

    cd.wait()

    def body_a(j, acc):
        return acc + _abshalf_tanh(chunk_v[pl.ds(j * L, L)])

    cr = lax.fori_loop(0, VPW, body_a, jnp.zeros((L,), jnp.float32))
    emask = (wid < NEXTRA).astype(jnp.float32)
    cr = cr + _abshalf_tanh(extra_v[...]) * emask

    for gd in gds:
        gd.wait()

    def body_b(j, carry):
        hl, cc = carry
        s = pl.ds(j * L, L)
        cal = 2.0 / (1.0 + jnp.exp(-gat_v[s]))
        cb = jnp.minimum(jnp.maximum(bel_v[s] * cal, 0.0), 1.0)
        cb_v[s] = cb
        g = gt_v[s]
        return hl + cb * (1.0 - g), cc + (1.0 - cb) * g

    zero = jnp.zeros((L,), jnp.float32)
    hl, cc = lax.fori_loop(0, BPW // L, body_b, (zero, zero))

    pltpu.sync_copy(cb_v, cb_out.at[pl.ds(b0, BPW)])

    part_v[...] = (hl * (1.0 / N_BATCH) + cc * (0.5 / N_BATCH)
                   + cr * (0.1 / N_AGENTS))
    pltpu.sync_copy(part_v, shared.at[pl.ds(sid * L, L)])
    plsc.subcore_barrier()

    @pl.when(sid == 0)
    def _():
        pltpu.sync_copy(shared, all_v)

        def body_r(s, acc):
            return acc + all_v[pl.ds(s * L, L)]

        acc = lax.fori_loop(0, NS, body_r, jnp.zeros((L,), jnp.float32))
        part_v[...] = acc
        pltpu.sync_copy(part_v, loss_out.at[cid])


def kernel(belief_strength, ground_truth, agent_ids, calibration_logits):
    ids = agent_ids.astype(jnp.int32).reshape(NW, NG, GCH)
    cb, loss_parts = _sc_doxastic(belief_strength, ground_truth, ids,
                                  calibration_logits)
    loss = jnp.sum(loss_parts)
    return (loss, cb)


# speedup vs baseline: 1.0590x; 1.0590x over previous
import functools

import jax
import jax.numpy as jnp
from jax import lax
from jax.experimental import pallas as pl
from jax.experimental.pallas import tpu as pltpu
from jax.experimental.pallas import tpu_sc as plsc

N_AGENTS = 1000000
N_BATCH = 16384
NC = 2
NS = 16
NW = NC * NS
L = 16

BPW = N_BATCH // NW
GCH = 128
NG = BPW // GCH
VPW = (N_AGENTS // L) // NW
MAIN = NW * VPW * L
NEXTRA = (N_AGENTS - MAIN) // L

_mesh = plsc.VectorSubcoreMesh(core_axis_name="c", subcore_axis_name="s")


@functools.partial(
    pl.kernel,
    mesh=_mesh,
    out_type=(
        jax.ShapeDtypeStruct((N_BATCH,), jnp.float32),
        jax.ShapeDtypeStruct((NC, L), jnp.float32),
    ),
    scratch_types=[
        pltpu.VMEM((NG, GCH), jnp.int32),
        pltpu.VMEM((BPW,), jnp.float32),
        pltpu.VMEM((BPW,), jnp.float32),
        pltpu.VMEM((BPW,), jnp.float32),
        pltpu.VMEM((BPW,), jnp.float32),
        pltpu.VMEM((VPW * L,), jnp.float32),
        pltpu.VMEM((L,), jnp.float32),
        pltpu.VMEM((L,), jnp.float32),
        pltpu.VMEM((NS * L,), jnp.float32),
        pltpu.VMEM_SHARED((NS * L,), jnp.float32),
        pltpu.SemaphoreType.DMA,
        pltpu.SemaphoreType.DMA,
    ],
)
def _sc_doxastic(bel_hbm, gt_hbm, ids_hbm, log_hbm, cb_out, loss_out,
                 idx_v, gat_v, bel_v, gt_v, cb_v, chunk_v, extra_v, part_v,
                 all_v, shared, sem_g, sem_c):
    cid = lax.axis_index("c")
    sid = lax.axis_index("s")
    wid = cid * NS + sid
    b0 = wid * BPW

    for k in range(NG):
        pltpu.sync_copy(ids_hbm.at[pl.ds(b0 + k * GCH, GCH)], idx_v.at[k])
    gds = [
        pltpu.async_copy(
            log_hbm.at[idx_v.at[k]],
            gat_v.at[pl.ds(k * GCH, GCH)],
            sem_g,
        )
        for k in range(NG)
    ]
    cd = pltpu.async_copy(log_hbm.at[pl.ds(wid * VPW * L, VPW * L)],
                          chunk_v, sem_c)
    pltpu.sync_copy(bel_hbm.at[pl.ds(b0, BPW)], bel_v)
    pltpu.sync_copy(gt_hbm.at[pl.ds(b0, BPW)], gt_v)
    eoff = MAIN + (wid % NEXTRA) * L
    pltpu.sync_copy(log_hbm.at[pl.ds(eoff, L)], extra_v)

    cd.wait()

    def q_of(x):
        return 1.0 / (1.0 + jnp.exp(jnp.minimum(x, -x)))

    def body_a(j, accs):
        a0, a1, a2 = accs
        base = j * (3 * L)
        a0 = a0 + q_of(chunk_v[pl.ds(base, L)])
        a1 = a1 + q_of(chunk_v[pl.ds(base + L, L)])
        a2 = a2 + q_of(chunk_v[pl.ds(base + 2 * L, L)])
        return a0, a1, a2

    zero = jnp.zeros((L,), jnp.float32)
    a0, a1, a2 = lax.fori_loop(0, VPW // 3, body_a, (zero, zero, zero))
    emask = (wid < NEXTRA).astype(jnp.float32)
    qe = 0.5 + (q_of(extra_v[...]) - 0.5) * emask
    cr = 2.0 * (a0 + a1 + a2 + qe) - float(VPW + 1)

    for gd in gds:
        gd.wait()

    def body_b(j, carry):
        hl, cc = carry
        s = pl.ds(j * L, L)
        cal = 2.0 / (1.0 + jnp.exp(-gat_v[s]))
        cb = jnp.minimum(jnp.maximum(bel_v[s] * cal, 0.0), 1.0)
        cb_v[s] = cb
        g = gt_v[s]
        return hl + cb * (1.0 - g), cc + (1.0 - cb) * g

    hl, cc = lax.fori_loop(0, BPW // L, body_b, (zero, zero))

    pltpu.sync_copy(cb_v, cb_out.at[pl.ds(b0, BPW)])

    part_v[...] = (hl * (1.0 / N_BATCH) + cc * (0.5 / N_BATCH)
                   + cr * (0.1 / N_AGENTS))
    pltpu.sync_copy(part_v, shared.at[pl.ds(sid * L, L)])
    plsc.subcore_barrier()

    @pl.when(sid == 0)
    def _():
        pltpu.sync_copy(shared, all_v)

        def body_r(s, acc):
            return acc + all_v[pl.ds(s * L, L)]

        acc = lax.fori_loop(0, NS, body_r, jnp.zeros((L,), jnp.float32))
        part_v[...] = acc
        pltpu.sync_copy(part_v, loss_out.at[cid])


def kernel(belief_strength, ground_truth, agent_ids, calibration_logits):
    ids = agent_ids.astype(jnp.int32)
    cb, loss_parts = _sc_doxastic(belief_strength, ground_truth, ids,
                                  calibration_logits)
    loss = jnp.sum(loss_parts)
    return (loss, cb)
